# baseline (speedup 1.0000x reference)
import functools

import jax
import jax.numpy as jnp
from jax import lax
from jax.experimental import pallas as pl
from jax.experimental.pallas import tpu as pltpu
from jax.experimental.pallas import tpu_sc as plsc

_DIM = 64
_SCALE = 8.0
_LANES = 16
_NC = 2
_NS = 16
_NW = _NC * _NS
_CHUNK = 128


def _sc_embed(idx, table):
    nchunks = idx.shape[0]
    cpw = nchunks // _NW
    mesh = plsc.VectorSubcoreMesh(core_axis_name="c", subcore_axis_name="s")

    @functools.partial(
        pl.kernel,
        mesh=mesh,
        out_type=jax.ShapeDtypeStruct((nchunks, _CHUNK, _DIM), jnp.float32),
        compiler_params=pltpu.CompilerParams(use_tc_tiling_on_sc=False),
        scratch_types=[
            pltpu.VMEM((cpw, _CHUNK), jnp.int32),
            pltpu.VMEM((_CHUNK, _DIM), jnp.float32),
            pltpu.SemaphoreType.DMA,
        ],
    )
    def body(idx_hbm, table_hbm, out_hbm, idx_v, rows_v, gsem):
        wid = lax.axis_index("s") * _NC + lax.axis_index("c")
        base = wid * cpw
        pltpu.sync_copy(idx_hbm.at[pl.ds(base, cpw)], idx_v)

        def chunk_body(j, carry):
            pltpu.async_copy(table_hbm.at[idx_v.at[j]], rows_v, gsem).wait()

            def mul_body(r, c2):
                for c in range(_DIM // _LANES):
                    sl = pl.ds(c * _LANES, _LANES)
                    rows_v[r, sl] = rows_v[r, sl] * _SCALE
                return c2

            lax.fori_loop(0, _CHUNK, mul_body, 0)
            pltpu.sync_copy(rows_v, out_hbm.at[base + j])
            return carry

        lax.fori_loop(0, cpw, chunk_body, 0)

    return body(idx, table)


def kernel(x, table):
    b, l = x.shape
    idx = x.reshape(b * l // _CHUNK, _CHUNK).astype(jnp.int32)
    out = _sc_embed(idx, table)
    return out.reshape(b, l, _DIM)


# speedup vs baseline: 1.1680x; 1.1680x over previous
import functools

import jax
import jax.numpy as jnp
from jax import lax
from jax.experimental import pallas as pl
from jax.experimental.pallas import tpu as pltpu
from jax.experimental.pallas import tpu_sc as plsc

_DIM = 64
_SCALE = 8.0
_LANES = 16
_NC = 2
_NS = 16
_NW = _NC * _NS
_CHUNK = 128


def _sc_embed(idx, table2):
    nchunks = idx.shape[0]
    cpw = nchunks // _NW
    mesh = plsc.VectorSubcoreMesh(core_axis_name="c", subcore_axis_name="s")

    @functools.partial(
        pl.kernel,
        mesh=mesh,
        out_type=jax.ShapeDtypeStruct((nchunks * _CHUNK, _DIM), jnp.float32),
        scratch_types=[
            pltpu.VMEM((cpw, _CHUNK), jnp.int32),
            pltpu.VMEM((cpw, _CHUNK), jnp.int32),
            pltpu.VMEM((_CHUNK, 2 * _DIM), jnp.float32),
            pltpu.VMEM((_CHUNK, 2 * _DIM), jnp.float32),
            pltpu.VMEM((_CHUNK, _DIM), jnp.float32),
            pltpu.VMEM((_CHUNK, _DIM), jnp.float32),
            pltpu.SemaphoreType.DMA,
            pltpu.SemaphoreType.DMA,
            pltpu.SemaphoreType.DMA,
            pltpu.SemaphoreType.DMA,
        ],
    )
    def body(idx_hbm, table_hbm, out_hbm, idx_v, hi_v, g0, g1, r0, r1,
             gs0, gs1, ss0, ss1):
        wid = lax.axis_index("s") * _NC + lax.axis_index("c")
        base = wid * cpw
        pltpu.sync_copy(idx_hbm.at[pl.ds(base, cpw)], idx_v)

        def hi_body(r, carry):
            for c in range(_CHUNK // _LANES):
                sl = pl.ds(c * _LANES, _LANES)
                hi_v[r, sl] = lax.shift_right_logical(idx_v[r, sl], 1)
            return carry

        lax.fori_loop(0, cpw, hi_body, 0)

        gbufs = (g0, g1)
        rbufs = (r0, r1)
        gsems = (gs0, gs1)
        ssems = (ss0, ss1)

        def gather_start(j, slot):
            pltpu.async_copy(table_hbm.at[hi_v.at[j]], gbufs[slot], gsems[slot])

        def gather_wait(slot):
            pltpu.make_async_copy(table_hbm.at[hi_v.at[0]], gbufs[slot],
                                  gsems[slot]).wait()

        def store_start(j, slot):
            pltpu.async_copy(rbufs[slot],
                             out_hbm.at[pl.ds((base + j) * _CHUNK, _CHUNK)],
                             ssems[slot])

        def store_wait(slot):
            pltpu.make_async_copy(rbufs[slot],
                                  out_hbm.at[pl.ds(0, _CHUNK)],
                                  ssems[slot]).wait()

        def compute(j, slot):
            g = gbufs[slot]
            r = rbufs[slot]

            def group_body(kk, carry):
                k0 = kk * _LANES
                par = (idx_v[j, pl.ds(k0, _LANES)] & 1) * _DIM
                for ll in range(_LANES):
                    h = par[ll]
                    for c in range(_DIM // _LANES):
                        src = g[k0 + ll, pl.ds(h + c * _LANES, _LANES)]
                        r[k0 + ll, pl.ds(c * _LANES, _LANES)] = src * _SCALE
                return carry

            lax.fori_loop(0, _CHUNK // _LANES, group_body, 0)

        gather_start(0, 0)
        gather_start(1, 1)

        def step(t, carry):
            for slot in range(2):
                j = 2 * t + slot
                gather_wait(slot)

                @pl.when(t > 0)
                def _():
                    store_wait(slot)

                compute(j, slot)

                @pl.when(j + 2 < cpw)
                def _():
                    gather_start(j + 2, slot)

                store_start(j, slot)
            return carry

        lax.fori_loop(0, cpw // 2, step, 0)
        store_wait(0)
        store_wait(1)

    return body(idx, table2)


def kernel(x, table):
    b, l = x.shape
    nrows = b * l
    idx = x.reshape(nrows // _CHUNK, _CHUNK).astype(jnp.int32)
    table2 = table.reshape(table.shape[0] // 2, 2 * _DIM)
    out = _sc_embed(idx, table2)
    return out.reshape(b, l, _DIM)
